# Initial kernel scaffold; baseline (speedup 1.0000x reference)
import jax
import jax.numpy as jnp
from jax.experimental import pallas as pl


def kernel(h, edge_index, b):
    raise NotImplementedError("write your pallas kernel here")


# speedup vs baseline: 4.1510x; 4.1510x over previous
import functools

import jax
import jax.numpy as jnp
from jax import lax
from jax.experimental import pallas as pl
from jax.experimental.pallas import tpu as pltpu
from jax.experimental.pallas import tpu_sc as plsc

N_NODES = 10000
N_EDGES = 320000
D_FEAT = 128

NC = 2
NS = 16
NW = NC * NS
LANES = 16

EPW = N_EDGES // NW
C = 80
NCH = EPW // C
GROUPS = C // LANES


def _tec_body(h_hbm, src_hbm, dst_hbm, b_hbm, out_hbm,
              b_v, idx_s, idx_d, rows_s, rows_d, out_v, sem_s, sem_d):
  wid = lax.axis_index("s") * NC + lax.axis_index("c")
  base = wid * EPW

  pltpu.sync_copy(b_hbm, b_v)

  lane = lax.iota(jnp.int32, LANES)

  def chunk(j, _):
    off = base + j * C
    pltpu.sync_copy(src_hbm.at[pl.ds(off, C)], idx_s)
    pltpu.sync_copy(dst_hbm.at[pl.ds(off, C)], idx_d)
    cs = pltpu.async_copy(h_hbm.at[idx_s], rows_s, sem_s)
    cd = pltpu.async_copy(h_hbm.at[idx_d], rows_d, sem_d)
    cs.wait()
    cd.wait()

    def group(g, _):
      row16 = g * LANES + lane
      acc = jnp.zeros((LANES,), jnp.float32)
      for f in range(D_FEAT):
        col = jnp.full((LANES,), f, jnp.int32)
        sv = plsc.load_gather(rows_s, [row16, col])
        dv = plsc.load_gather(rows_d, [row16, col])
        acc = acc + sv * dv
      si = idx_s[pl.ds(g * LANES, LANES)]
      di = idx_d[pl.ds(g * LANES, LANES)]
      acc = acc + plsc.load_gather(b_v, [si]) + plsc.load_gather(b_v, [di])
      out_v[pl.ds(g * LANES, LANES)] = acc
      return 0

    lax.fori_loop(0, GROUPS, group, 0)
    pltpu.sync_copy(out_v, out_hbm.at[pl.ds(off, C)])
    return 0

  lax.fori_loop(0, NCH, chunk, 0)


@functools.partial(jax.jit, static_argnames=())
def _run(h, src, dst, b_flat):
  mesh = plsc.VectorSubcoreMesh(
      core_axis_name="c", subcore_axis_name="s", num_cores=NC, num_subcores=NS)
  fn = pl.kernel(
      _tec_body,
      out_type=jax.ShapeDtypeStruct((N_EDGES,), jnp.float32),
      mesh=mesh,
      scratch_types=[
          pltpu.VMEM((N_NODES,), jnp.float32),
          pltpu.VMEM((C,), jnp.int32),
          pltpu.VMEM((C,), jnp.int32),
          pltpu.VMEM((C, D_FEAT), jnp.float32),
          pltpu.VMEM((C, D_FEAT), jnp.float32),
          pltpu.VMEM((C,), jnp.float32),
          pltpu.SemaphoreType.DMA,
          pltpu.SemaphoreType.DMA,
      ],
      compiler_params=pltpu.CompilerParams(needs_layout_passes=False),
  )
  return fn(h, src, dst, b_flat)


def kernel(h, edge_index, b):
  src = edge_index[0].astype(jnp.int32)
  dst = edge_index[1].astype(jnp.int32)
  out = _run(h, src, dst, b[:, 0])
  return out.reshape(N_EDGES, 1)


# speedup vs baseline: 5.0609x; 1.2192x over previous
import functools

import jax
import jax.numpy as jnp
from jax import lax
from jax.experimental import pallas as pl
from jax.experimental.pallas import tpu as pltpu
from jax.experimental.pallas import tpu_sc as plsc

N_NODES = 10000
N_EDGES = 320000
D_FEAT = 128

NC = 2
NS = 16
NW = NC * NS
LANES = 16

EPW = N_EDGES // NW
C = 80
NCH = EPW // C
GROUPS = C // LANES


def _tec_body(h_hbm, src_hbm, dst_hbm, b_hbm, out_hbm,
              b_v, isa, ida, out_v, rsa, rda, rsb, rdb,
              sem_sa, sem_da, sem_sb, sem_db):
  wid = lax.axis_index("s") * NC + lax.axis_index("c")
  base = wid * EPW

  pltpu.sync_copy(b_hbm, b_v)
  pltpu.sync_copy(src_hbm.at[pl.ds(base, EPW)], isa)
  pltpu.sync_copy(dst_hbm.at[pl.ds(base, EPW)], ida)

  lane = lax.iota(jnp.int32, LANES)

  def start(j, rs, rd, sem_s, sem_d):
    s = pltpu.async_copy(h_hbm.at[isa.at[pl.ds(j * C, C)]], rs, sem_s)
    d = pltpu.async_copy(h_hbm.at[ida.at[pl.ds(j * C, C)]], rd, sem_d)
    return s, d

  def wait(j, rs, rd, sem_s, sem_d):
    pltpu.make_async_copy(h_hbm.at[isa.at[pl.ds(j * C, C)]], rs, sem_s).wait()
    pltpu.make_async_copy(h_hbm.at[ida.at[pl.ds(j * C, C)]], rd, sem_d).wait()

  def compute(j, rs, rd):
    def group(g, _):
      e0 = j * C + g * LANES
      row16 = g * LANES + lane
      acc = jnp.zeros((LANES,), jnp.float32)
      for f in range(D_FEAT):
        col = jnp.full((LANES,), f, jnp.int32)
        sv = plsc.load_gather(rs, [row16, col])
        dv = plsc.load_gather(rd, [row16, col])
        acc = acc + sv * dv
      si = isa[pl.ds(e0, LANES)]
      di = ida[pl.ds(e0, LANES)]
      acc = acc + plsc.load_gather(b_v, [si]) + plsc.load_gather(b_v, [di])
      out_v[pl.ds(e0, LANES)] = acc
      return 0

    lax.fori_loop(0, GROUPS, group, 0)

  start(0, rsa, rda, sem_sa, sem_da)

  def body(t, _):
    j = 2 * t
    start(j + 1, rsb, rdb, sem_sb, sem_db)
    wait(j, rsa, rda, sem_sa, sem_da)
    compute(j, rsa, rda)
    start(j + 2, rsa, rda, sem_sa, sem_da)
    wait(j + 1, rsb, rdb, sem_sb, sem_db)
    compute(j + 1, rsb, rdb)
    return 0

  lax.fori_loop(0, NCH // 2, body, 0)
  wait(NCH - 1, rsa, rda, sem_sa, sem_da)
  compute(NCH - 1, rsa, rda)

  pltpu.sync_copy(out_v, out_hbm.at[pl.ds(base, EPW)])


@jax.jit
def _run(h, src, dst, b_flat):
  mesh = plsc.VectorSubcoreMesh(
      core_axis_name="c", subcore_axis_name="s", num_cores=NC, num_subcores=NS)
  fn = pl.kernel(
      _tec_body,
      out_type=jax.ShapeDtypeStruct((N_EDGES,), jnp.float32),
      mesh=mesh,
      scratch_types=[
          pltpu.VMEM((N_NODES,), jnp.float32),
          pltpu.VMEM((EPW,), jnp.int32),
          pltpu.VMEM((EPW,), jnp.int32),
          pltpu.VMEM((EPW,), jnp.float32),
          pltpu.VMEM((C, D_FEAT), jnp.float32),
          pltpu.VMEM((C, D_FEAT), jnp.float32),
          pltpu.VMEM((C, D_FEAT), jnp.float32),
          pltpu.VMEM((C, D_FEAT), jnp.float32),
          pltpu.SemaphoreType.DMA,
          pltpu.SemaphoreType.DMA,
          pltpu.SemaphoreType.DMA,
          pltpu.SemaphoreType.DMA,
      ],
      compiler_params=pltpu.CompilerParams(needs_layout_passes=False),
  )
  return fn(h, src, dst, b_flat)


def kernel(h, edge_index, b):
  src = edge_index[0].astype(jnp.int32)
  dst = edge_index[1].astype(jnp.int32)
  out = _run(h, src, dst, b[:, 0])
  return out.reshape(N_EDGES, 1)


# speedup vs baseline: 16.1023x; 3.1817x over previous
import functools

import jax
import jax.numpy as jnp
from jax import lax
from jax.experimental import pallas as pl
from jax.experimental.pallas import tpu as pltpu
from jax.experimental.pallas import tpu_sc as plsc

N_NODES = 10000
N_EDGES = 320000
D_FEAT = 128

NC = 2
NS = 16
NW = NC * NS
LANES = 16

EPW = N_EDGES // NW
C = 80
NCH = EPW // C
GROUPS = C // LANES


def _tec_body(h_hbm, src_hbm, dst_hbm, b_hbm, out_hbm,
              b_v, isa, ida, out_v, rsa, rda, rsb, rdb,
              sem_sa, sem_da, sem_sb, sem_db):
  wid = lax.axis_index("s") * NC + lax.axis_index("c")
  base = wid * EPW

  pltpu.sync_copy(b_hbm, b_v)
  pltpu.sync_copy(src_hbm.at[pl.ds(base, EPW)], isa)
  pltpu.sync_copy(dst_hbm.at[pl.ds(base, EPW)], ida)

  lane = lax.iota(jnp.int32, LANES)

  def start(j, rs, rd, sem_s, sem_d):
    s = pltpu.async_copy(h_hbm.at[isa.at[pl.ds(j * C, C)]], rs, sem_s)
    d = pltpu.async_copy(h_hbm.at[ida.at[pl.ds(j * C, C)]], rd, sem_d)
    return s, d

  def wait(j, rs, rd, sem_s, sem_d):
    pltpu.make_async_copy(h_hbm.at[isa.at[pl.ds(j * C, C)]], rs, sem_s).wait()
    pltpu.make_async_copy(h_hbm.at[ida.at[pl.ds(j * C, C)]], rd, sem_d).wait()

  def compute(j, rs, rd):
    def group(g, _):
      e0 = j * C + g * LANES
      row16 = g * LANES + lane
      acc = jnp.zeros((LANES,), jnp.float32)
      for f in range(D_FEAT):
        col = (lane + f) & (D_FEAT - 1)
        sv = plsc.load_gather(rs, [row16, col])
        dv = plsc.load_gather(rd, [row16, col])
        acc = acc + sv * dv
      si = isa[pl.ds(e0, LANES)]
      di = ida[pl.ds(e0, LANES)]
      acc = acc + plsc.load_gather(b_v, [si]) + plsc.load_gather(b_v, [di])
      out_v[pl.ds(e0, LANES)] = acc
      return 0

    lax.fori_loop(0, GROUPS, group, 0)

  start(0, rsa, rda, sem_sa, sem_da)

  def body(t, _):
    j = 2 * t
    start(j + 1, rsb, rdb, sem_sb, sem_db)
    wait(j, rsa, rda, sem_sa, sem_da)
    compute(j, rsa, rda)
    start(j + 2, rsa, rda, sem_sa, sem_da)
    wait(j + 1, rsb, rdb, sem_sb, sem_db)
    compute(j + 1, rsb, rdb)
    return 0

  lax.fori_loop(0, NCH // 2, body, 0)
  wait(NCH - 1, rsa, rda, sem_sa, sem_da)
  compute(NCH - 1, rsa, rda)

  pltpu.sync_copy(out_v, out_hbm.at[pl.ds(base, EPW)])


@jax.jit
def _run(h, src, dst, b_flat):
  mesh = plsc.VectorSubcoreMesh(
      core_axis_name="c", subcore_axis_name="s", num_cores=NC, num_subcores=NS)
  fn = pl.kernel(
      _tec_body,
      out_type=jax.ShapeDtypeStruct((N_EDGES,), jnp.float32),
      mesh=mesh,
      scratch_types=[
          pltpu.VMEM((N_NODES,), jnp.float32),
          pltpu.VMEM((EPW,), jnp.int32),
          pltpu.VMEM((EPW,), jnp.int32),
          pltpu.VMEM((EPW,), jnp.float32),
          pltpu.VMEM((C, D_FEAT), jnp.float32),
          pltpu.VMEM((C, D_FEAT), jnp.float32),
          pltpu.VMEM((C, D_FEAT), jnp.float32),
          pltpu.VMEM((C, D_FEAT), jnp.float32),
          pltpu.SemaphoreType.DMA,
          pltpu.SemaphoreType.DMA,
          pltpu.SemaphoreType.DMA,
          pltpu.SemaphoreType.DMA,
      ],
      compiler_params=pltpu.CompilerParams(needs_layout_passes=False),
  )
  return fn(h, src, dst, b_flat)


def kernel(h, edge_index, b):
  src = edge_index[0].astype(jnp.int32)
  dst = edge_index[1].astype(jnp.int32)
  out = _run(h, src, dst, b[:, 0])
  return out.reshape(N_EDGES, 1)
